# baseline (speedup 1.0000x reference)
import jax
import jax.numpy as jnp
from jax.experimental import pallas as pl
from jax.experimental.pallas import tpu as pltpu

_SIGMA = 0.01
_TOTAL_COUNT = 10
_BLOCK_COLS = 4096


def _add_scalar_kernel(c_ref, x_ref, o_ref):
    o_ref[...] = x_ref[...] + c_ref[0]


def kernel(X):
    k = jax.random.fold_in(jax.random.key(0), 1)
    logits = jnp.log(jnp.full((4,), 0.25, dtype=jnp.float32))
    draws = jax.random.categorical(k, logits, shape=(_TOTAL_COUNT,))
    noise = (_SIGMA * jnp.sum(draws == 0).astype(X.dtype)).reshape(1)

    rows, cols = X.shape
    grid = (pl.cdiv(cols, _BLOCK_COLS),)
    return pl.pallas_call(
        _add_scalar_kernel,
        grid=grid,
        in_specs=[
            pl.BlockSpec(memory_space=pltpu.SMEM),
            pl.BlockSpec((rows, _BLOCK_COLS), lambda j: (0, j)),
        ],
        out_specs=pl.BlockSpec((rows, _BLOCK_COLS), lambda j: (0, j)),
        out_shape=jax.ShapeDtypeStruct((rows, cols), X.dtype),
        compiler_params=pltpu.CompilerParams(
            dimension_semantics=("arbitrary",),
        ),
    )(noise, X)


# speedup vs baseline: 1.0157x; 1.0157x over previous
import jax
import jax.numpy as jnp
from jax.experimental import pallas as pl
from jax.experimental.pallas import tpu as pltpu

_SIGMA = 0.01
_TOTAL_COUNT = 10
_BLOCK_ROWS = 8


def _add_scalar_kernel(c_ref, x_ref, o_ref):
    o_ref[...] = x_ref[...] + c_ref[0]


def kernel(X):
    k = jax.random.fold_in(jax.random.key(0), 1)
    logits = jnp.log(jnp.full((4,), 0.25, dtype=jnp.float32))
    draws = jax.random.categorical(k, logits, shape=(_TOTAL_COUNT,))
    noise = (_SIGMA * jnp.sum(draws == 0).astype(X.dtype)).reshape(1)

    rows, cols = X.shape
    grid = (pl.cdiv(rows, _BLOCK_ROWS),)
    return pl.pallas_call(
        _add_scalar_kernel,
        grid=grid,
        in_specs=[
            pl.BlockSpec(memory_space=pltpu.SMEM),
            pl.BlockSpec((_BLOCK_ROWS, cols), lambda j: (j, 0)),
        ],
        out_specs=pl.BlockSpec((_BLOCK_ROWS, cols), lambda j: (j, 0)),
        out_shape=jax.ShapeDtypeStruct((rows, cols), X.dtype),
        compiler_params=pltpu.CompilerParams(
            dimension_semantics=("arbitrary",),
        ),
    )(noise, X)


# speedup vs baseline: 1.0331x; 1.0172x over previous
import jax
import jax.numpy as jnp
from jax.experimental import pallas as pl
from jax.experimental.pallas import tpu as pltpu

_SIGMA = 0.01
_TOTAL_COUNT = 10
_BLOCK_ROWS = 32


def _add_scalar_kernel(c_ref, x_ref, o_ref):
    o_ref[...] = x_ref[...] + c_ref[0]


def kernel(X):
    k = jax.random.fold_in(jax.random.key(0), 1)
    logits = jnp.log(jnp.full((4,), 0.25, dtype=jnp.float32))
    draws = jax.random.categorical(k, logits, shape=(_TOTAL_COUNT,))
    noise = (_SIGMA * jnp.sum(draws == 0).astype(X.dtype)).reshape(1)

    rows, cols = X.shape
    grid = (pl.cdiv(rows, _BLOCK_ROWS),)
    return pl.pallas_call(
        _add_scalar_kernel,
        grid=grid,
        in_specs=[
            pl.BlockSpec(memory_space=pltpu.SMEM),
            pl.BlockSpec((_BLOCK_ROWS, cols), lambda j: (j, 0)),
        ],
        out_specs=pl.BlockSpec((_BLOCK_ROWS, cols), lambda j: (j, 0)),
        out_shape=jax.ShapeDtypeStruct((rows, cols), X.dtype),
        compiler_params=pltpu.CompilerParams(
            dimension_semantics=("arbitrary",),
        ),
    )(noise, X)
